# baseline (speedup 1.0000x reference)
import functools

import jax
import jax.numpy as jnp
from jax import lax
from jax.experimental import pallas as pl
from jax.experimental.pallas import tpu as pltpu
from jax.experimental.pallas import tpu_sc as plsc

_VOCAB = 100000
_D = 1024
_B = 4
_S = 8192
_SCALE = 32.0
_EPS = 1e-5

_L = 16
_NVEC = _D // _L
_R = _S * _B
_NW = 32
_RPW = _R // _NW
_C = 16
_NCH = _RPW // _C
_U = 8
_NBUF = 2


def _rsqrt_v(x):
    i = lax.bitcast_convert_type(x, jnp.int32)
    i = jnp.int32(0x5F3759DF) - lax.shift_right_arithmetic(i, 1)
    y = lax.bitcast_convert_type(i, jnp.float32)
    for _ in range(3):
        y = y * (1.5 - 0.5 * x * y * y)
    return y


_GATHER_DNUMS = lax.GatherDimensionNumbers(
    offset_dims=(), collapsed_slice_dims=(0,), start_index_map=(0,))


def _lane_sum(s):
    io = lax.iota(jnp.int32, 16)
    for k in (8, 4, 2, 1):
        perm = lax.gather(s, (io ^ k)[:, None], _GATHER_DNUMS, (1,),
                          mode=lax.GatherScatterMode.PROMISE_IN_BOUNDS)
        s = s + perm
    return s


def _emb_body(idx_hbm, tok_hbm, pos_hbm, gam_hbm, bet_hbm, out_hbm,
              idx_v, in_v, out_v, pos_v, gam_v, bet_v,
              gsem0, gsem1, psem0, psem1, wsem0, wsem1):
    wid = lax.axis_index("s") * 2 + lax.axis_index("c")
    base = wid * _RPW
    gsems = (gsem0, gsem1)
    psems = (psem0, psem1)
    wsems = (wsem0, wsem1)

    pltpu.sync_copy(idx_hbm.at[pl.ds(pl.multiple_of(base, _RPW), _RPW)], idx_v)
    pltpu.sync_copy(gam_hbm, gam_v)
    pltpu.sync_copy(bet_hbm, bet_v)

    def start_fetch(g, b):
        row0 = pl.multiple_of(base + g * _C, _C)
        off = pl.ds(pl.multiple_of(g * _C, _C), _C)
        pltpu.make_async_copy(
            tok_hbm.at[idx_v.at[off]], in_v.at[b], gsems[b]).start()
        pos0 = pl.multiple_of(row0 // _B, _C // _B)
        pltpu.make_async_copy(
            pos_hbm.at[pl.ds(pos0, _C // _B)], pos_v.at[b], psems[b]).start()

    def wait_fetch(b):
        pltpu.make_async_copy(
            tok_hbm.at[idx_v.at[pl.ds(0, _C)]], in_v.at[b], gsems[b]).wait()
        pltpu.make_async_copy(
            pos_hbm.at[pl.ds(0, _C // _B)], pos_v.at[b], psems[b]).wait()

    def wait_wb(b):
        pltpu.make_async_copy(
            out_v.at[b], out_hbm.at[pl.ds(0, _C)], wsems[b]).wait()

    def compute(b):
        inb = in_v.at[b]
        outb = out_v.at[b]
        posb = pos_v.at[b]

        def quad_body(q, _):
            i0 = q * _B

            def acc(jj, carry):
                ss = list(carry)
                for u in range(_U):
                    o = pl.ds(pl.multiple_of(jj * (_U * _L) + u * _L, _L), _L)
                    p = posb[q, o]
                    for r in range(_B):
                        y = inb[i0 + r, o] * _SCALE + p
                        outb[i0 + r, o] = y
                        ss[r] = ss[r] + y
                        ss[_B + r] = ss[_B + r] + y * y
                return tuple(ss)

            zero = jnp.zeros((_L,), jnp.float32)
            carry = lax.fori_loop(0, _NVEC // _U, acc, (zero,) * (2 * _B))
            means = [_lane_sum(carry[r]) * (1.0 / _D) for r in range(_B)]
            invs = [
                _rsqrt_v(_lane_sum(carry[_B + r]) * (1.0 / _D)
                         - means[r] * means[r] + _EPS)
                for r in range(_B)
            ]

            def norm(jj, _):
                for u in range(_U):
                    o = pl.ds(pl.multiple_of(jj * (_U * _L) + u * _L, _L), _L)
                    gmm = gam_v[o]
                    bt = bet_v[o]
                    for r in range(_B):
                        y = (outb[i0 + r, o] - means[r]) * invs[r]
                        outb[i0 + r, o] = y * gmm + bt
                return 0

            lax.fori_loop(0, _NVEC // _U, norm, 0)
            return 0

        lax.fori_loop(0, _C // _B, quad_body, 0)

    def chunk(g, b, wait_writeback, prefetch):
        wait_fetch(b)
        if wait_writeback:
            wait_wb(b)
        compute(b)
        row0 = pl.multiple_of(base + g * _C, _C)
        pltpu.make_async_copy(
            out_v.at[b], out_hbm.at[pl.ds(row0, _C)], wsems[b]).start()
        if prefetch:
            start_fetch(g + _NBUF, b)

    start_fetch(0, 0)
    start_fetch(1, 1)
    chunk(0, 0, wait_writeback=False, prefetch=True)
    chunk(1, 1, wait_writeback=False, prefetch=True)

    def round_body(i, _):
        g0 = i * _NBUF
        chunk(g0, 0, wait_writeback=True, prefetch=True)
        chunk(g0 + 1, 1, wait_writeback=True, prefetch=True)
        return 0

    lax.fori_loop(1, _NCH // _NBUF - 1, round_body, 0)
    chunk(_NCH - 2, 0, wait_writeback=True, prefetch=False)
    chunk(_NCH - 1, 1, wait_writeback=True, prefetch=False)
    wait_wb(0)
    wait_wb(1)


_emb_kernel = functools.partial(
    pl.kernel,
    mesh=plsc.VectorSubcoreMesh(core_axis_name="c", subcore_axis_name="s"),
    out_type=jax.ShapeDtypeStruct((_R, _D), jnp.float32),
    scratch_types=[
        pltpu.VMEM((_RPW,), jnp.int32),
        pltpu.VMEM((_NBUF, _C, _D), jnp.float32),
        pltpu.VMEM((_NBUF, _C, _D), jnp.float32),
        pltpu.VMEM((_NBUF, _C // _B, _D), jnp.float32),
        pltpu.VMEM((_D,), jnp.float32),
        pltpu.VMEM((_D,), jnp.float32),
        pltpu.SemaphoreType.DMA,
        pltpu.SemaphoreType.DMA,
        pltpu.SemaphoreType.DMA,
        pltpu.SemaphoreType.DMA,
        pltpu.SemaphoreType.DMA,
        pltpu.SemaphoreType.DMA,
    ],
)(_emb_body)


def kernel(src_tokens, tok_table, pos_table, ln_gamma, ln_beta):
    idx = src_tokens.T.reshape(-1)
    out = _emb_kernel(idx, tok_table, pos_table, ln_gamma, ln_beta)
    return out.reshape(_S, _B, _D)


# speedup vs baseline: 1.7347x; 1.7347x over previous
import functools

import jax
import jax.numpy as jnp
from jax import lax
from jax.experimental import pallas as pl
from jax.experimental.pallas import tpu as pltpu
from jax.experimental.pallas import tpu_sc as plsc

_VOCAB = 100000
_D = 1024
_B = 4
_S = 8192
_SCALE = 32.0
_EPS = 1e-5

_R = _S * _B
_NW = 32
_RPW = _R // _NW
_C = 16
_NCH = _RPW // _C
_NBUF = 4

_RB = 512


def _gather_body(idx_hbm, tok_hbm, out_hbm,
                 idx_v, buf_v,
                 gsem0, gsem1, gsem2, gsem3, wsem0, wsem1, wsem2, wsem3):
    wid = lax.axis_index("s") * 2 + lax.axis_index("c")
    base = wid * _RPW
    gsems = (gsem0, gsem1, gsem2, gsem3)
    wsems = (wsem0, wsem1, wsem2, wsem3)

    pltpu.sync_copy(idx_hbm.at[pl.ds(pl.multiple_of(base, _RPW), _RPW)], idx_v)

    def start_fetch(g, b):
        off = pl.ds(pl.multiple_of(g * _C, _C), _C)
        pltpu.make_async_copy(
            tok_hbm.at[idx_v.at[off]], buf_v.at[b], gsems[b]).start()

    def wait_fetch(b):
        pltpu.make_async_copy(
            tok_hbm.at[idx_v.at[pl.ds(0, _C)]], buf_v.at[b], gsems[b]).wait()

    def start_wb(g, b):
        row0 = pl.multiple_of(base + g * _C, _C)
        pltpu.make_async_copy(
            buf_v.at[b], out_hbm.at[pl.ds(row0, _C)], wsems[b]).start()

    def wait_wb(b):
        pltpu.make_async_copy(
            buf_v.at[b], out_hbm.at[pl.ds(0, _C)], wsems[b]).wait()

    def chunk(g, b, wait_writeback, prefetch):
        wait_fetch(b)
        start_wb(g, b)
        b2 = (b + 2) % _NBUF
        if wait_writeback:
            wait_wb(b2)
        if prefetch:
            start_fetch(g + 2, b2)

    start_fetch(0, 0)
    start_fetch(1, 1)
    chunk(0, 0, wait_writeback=False, prefetch=True)
    chunk(1, 1, wait_writeback=False, prefetch=True)
    chunk(2, 2, wait_writeback=True, prefetch=True)
    chunk(3, 3, wait_writeback=True, prefetch=True)

    def round_body(i, _):
        g0 = i * _NBUF
        chunk(g0, 0, wait_writeback=True, prefetch=True)
        chunk(g0 + 1, 1, wait_writeback=True, prefetch=True)
        chunk(g0 + 2, 2, wait_writeback=True, prefetch=True)
        chunk(g0 + 3, 3, wait_writeback=True, prefetch=True)
        return 0

    lax.fori_loop(1, _NCH // _NBUF - 1, round_body, 0)
    chunk(_NCH - 4, 0, wait_writeback=True, prefetch=True)
    chunk(_NCH - 3, 1, wait_writeback=True, prefetch=True)
    chunk(_NCH - 2, 2, wait_writeback=False, prefetch=False)
    chunk(_NCH - 1, 3, wait_writeback=False, prefetch=False)
    wait_wb(0)
    wait_wb(1)
    wait_wb(2)
    wait_wb(3)


_gather_kernel = functools.partial(
    pl.kernel,
    mesh=plsc.VectorSubcoreMesh(core_axis_name="c", subcore_axis_name="s"),
    out_type=jax.ShapeDtypeStruct((_R, _D), jnp.float32),
    scratch_types=[
        pltpu.VMEM((_RPW,), jnp.int32),
        pltpu.VMEM((_NBUF, _C, _D), jnp.float32),
        pltpu.SemaphoreType.DMA,
        pltpu.SemaphoreType.DMA,
        pltpu.SemaphoreType.DMA,
        pltpu.SemaphoreType.DMA,
        pltpu.SemaphoreType.DMA,
        pltpu.SemaphoreType.DMA,
        pltpu.SemaphoreType.DMA,
        pltpu.SemaphoreType.DMA,
    ],
)(_gather_body)


def _ln_body(g_ref, p_ref, gam_ref, bet_ref, o_ref):
    g = g_ref[...]
    p = p_ref[...]
    y = g.reshape(_RB // _B, _B, _D) * _SCALE + p[:, None, :]
    mean = jnp.mean(y, axis=-1, keepdims=True)
    yc = y - mean
    var = jnp.mean(yc * yc, axis=-1, keepdims=True)
    out = yc * lax.rsqrt(var + _EPS) * gam_ref[0] + bet_ref[0]
    o_ref[...] = out.reshape(_RB, _D)


_ln_call = pl.pallas_call(
    _ln_body,
    grid=(_R // _RB,),
    in_specs=[
        pl.BlockSpec((_RB, _D), lambda i: (i, 0)),
        pl.BlockSpec((_RB // _B, _D), lambda i: (i, 0)),
        pl.BlockSpec((1, _D), lambda i: (0, 0)),
        pl.BlockSpec((1, _D), lambda i: (0, 0)),
    ],
    out_specs=pl.BlockSpec((_RB, _D), lambda i: (i, 0)),
    out_shape=jax.ShapeDtypeStruct((_R, _D), jnp.float32),
)


def kernel(src_tokens, tok_table, pos_table, ln_gamma, ln_beta):
    idx = src_tokens.T.reshape(-1)
    gathered = _gather_kernel(idx, tok_table)
    out = _ln_call(gathered, pos_table,
                   ln_gamma.reshape(1, _D), ln_beta.reshape(1, _D))
    return out.reshape(_S, _B, _D)


# speedup vs baseline: 1.7558x; 1.0122x over previous
import functools

import jax
import jax.numpy as jnp
from jax import lax
from jax.experimental import pallas as pl
from jax.experimental.pallas import tpu as pltpu
from jax.experimental.pallas import tpu_sc as plsc

_VOCAB = 100000
_D = 1024
_B = 4
_S = 8192
_SCALE = 32.0
_EPS = 1e-5

_R = _S * _B
_NW = 32
_RPW = _R // _NW
_C = 16
_NCH = _RPW // _C
_NBUF = 4

_RB = 512


def _gather_body(idx_hbm, tok_hbm, out_hbm,
                 idx_v, buf_v,
                 gsem0, gsem1, gsem2, gsem3, wsem0, wsem1, wsem2, wsem3):
    wid = lax.axis_index("s") * 2 + lax.axis_index("c")
    base = wid * _RPW
    gsems = (gsem0, gsem1, gsem2, gsem3)
    wsems = (wsem0, wsem1, wsem2, wsem3)

    pltpu.sync_copy(idx_hbm.at[pl.ds(pl.multiple_of(base, _RPW), _RPW)], idx_v)

    def start_fetch(g, b):
        off = pl.ds(pl.multiple_of(g * _C, _C), _C)
        pltpu.make_async_copy(
            tok_hbm.at[idx_v.at[off]], buf_v.at[b], gsems[b]).start()

    def wait_fetch(b):
        pltpu.make_async_copy(
            tok_hbm.at[idx_v.at[pl.ds(0, _C)]], buf_v.at[b], gsems[b]).wait()

    def start_wb(g, b):
        row0 = pl.multiple_of(base + g * _C, _C)
        pltpu.make_async_copy(
            buf_v.at[b], out_hbm.at[pl.ds(row0, _C)], wsems[b]).start()

    def wait_wb(b):
        pltpu.make_async_copy(
            buf_v.at[b], out_hbm.at[pl.ds(0, _C)], wsems[b]).wait()

    def chunk(g, b, wait_writeback, prefetch):
        wait_fetch(b)
        start_wb(g, b)
        b2 = (b + 2) % _NBUF
        if wait_writeback:
            wait_wb(b2)
        if prefetch:
            start_fetch(g + 2, b2)

    start_fetch(0, 0)
    start_fetch(1, 1)
    chunk(0, 0, wait_writeback=False, prefetch=True)
    chunk(1, 1, wait_writeback=False, prefetch=True)
    chunk(2, 2, wait_writeback=True, prefetch=True)
    chunk(3, 3, wait_writeback=True, prefetch=True)

    def round_body(i, _):
        g0 = i * _NBUF
        chunk(g0, 0, wait_writeback=True, prefetch=True)
        chunk(g0 + 1, 1, wait_writeback=True, prefetch=True)
        chunk(g0 + 2, 2, wait_writeback=True, prefetch=True)
        chunk(g0 + 3, 3, wait_writeback=True, prefetch=True)
        return 0

    lax.fori_loop(1, _NCH // _NBUF - 1, round_body, 0)
    chunk(_NCH - 4, 0, wait_writeback=True, prefetch=True)
    chunk(_NCH - 3, 1, wait_writeback=True, prefetch=True)
    chunk(_NCH - 2, 2, wait_writeback=False, prefetch=False)
    chunk(_NCH - 1, 3, wait_writeback=False, prefetch=False)
    wait_wb(0)
    wait_wb(1)
    wait_wb(2)
    wait_wb(3)


_gather_kernel = functools.partial(
    pl.kernel,
    mesh=plsc.VectorSubcoreMesh(core_axis_name="c", subcore_axis_name="s"),
    out_type=jax.ShapeDtypeStruct((_R, _D), jnp.float32),
    scratch_types=[
        pltpu.VMEM((_RPW,), jnp.int32),
        pltpu.VMEM((_NBUF, _C, _D), jnp.float32),
        pltpu.SemaphoreType.DMA,
        pltpu.SemaphoreType.DMA,
        pltpu.SemaphoreType.DMA,
        pltpu.SemaphoreType.DMA,
        pltpu.SemaphoreType.DMA,
        pltpu.SemaphoreType.DMA,
        pltpu.SemaphoreType.DMA,
        pltpu.SemaphoreType.DMA,
    ],
)(_gather_body)


def _ln_body(g_ref, p_ref, gam_ref, bet_ref, o_ref):
    y = g_ref[...] * _SCALE + p_ref[...]
    mean = jnp.mean(y, axis=-1, keepdims=True)
    yc = y - mean
    var = jnp.mean(yc * yc, axis=-1, keepdims=True)
    o_ref[...] = yc * lax.rsqrt(var + _EPS) * gam_ref[0] + bet_ref[0]


_SBLK = _S // _RB

_ln_call = pl.pallas_call(
    _ln_body,
    grid=(_R // _RB,),
    in_specs=[
        pl.BlockSpec((_RB, _D), lambda i: ((i % _B) * _SBLK + i // _B, 0)),
        pl.BlockSpec((_RB, _D), lambda i: (i // _B, 0)),
        pl.BlockSpec((1, _D), lambda i: (0, 0)),
        pl.BlockSpec((1, _D), lambda i: (0, 0)),
    ],
    out_specs=pl.BlockSpec((_RB, _D), lambda i: (i // _B, i % _B)),
    out_shape=jax.ShapeDtypeStruct((_S, _B * _D), jnp.float32),
    compiler_params=pltpu.CompilerParams(
        dimension_semantics=("parallel",)),
)


def kernel(src_tokens, tok_table, pos_table, ln_gamma, ln_beta):
    idx = src_tokens.reshape(-1)
    gathered = _gather_kernel(idx, tok_table)
    out = _ln_call(gathered, pos_table,
                   ln_gamma.reshape(1, _D), ln_beta.reshape(1, _D))
    return out.reshape(_S, _B, _D)


# speedup vs baseline: 1.8292x; 1.0418x over previous
import functools

import jax
import jax.numpy as jnp
from jax import lax
from jax.experimental import pallas as pl
from jax.experimental.pallas import tpu as pltpu
from jax.experimental.pallas import tpu_sc as plsc

_VOCAB = 100000
_D = 1024
_B = 4
_S = 8192
_SCALE = 32.0
_EPS = 1e-5

_R = _S * _B
_NW = 32
_RPW = _R // _NW
_C = 16
_NCH = _RPW // _C
_NBUF = 4

_RB = 1024


def _gather_body(idx_hbm, tok_hbm, out_hbm,
                 idx_v, buf_v,
                 gsem0, gsem1, gsem2, gsem3, wsem0, wsem1, wsem2, wsem3):
    wid = lax.axis_index("s") * 2 + lax.axis_index("c")
    base = wid * _RPW
    gsems = (gsem0, gsem1, gsem2, gsem3)
    wsems = (wsem0, wsem1, wsem2, wsem3)

    pltpu.sync_copy(idx_hbm.at[pl.ds(pl.multiple_of(base, _RPW), _RPW)], idx_v)

    def start_fetch(g, b):
        off = pl.ds(pl.multiple_of(g * _C, _C), _C)
        pltpu.make_async_copy(
            tok_hbm.at[idx_v.at[off]], buf_v.at[b], gsems[b]).start()

    def wait_fetch(b):
        pltpu.make_async_copy(
            tok_hbm.at[idx_v.at[pl.ds(0, _C)]], buf_v.at[b], gsems[b]).wait()

    def start_wb(g, b):
        row0 = pl.multiple_of(base + g * _C, _C)
        pltpu.make_async_copy(
            buf_v.at[b], out_hbm.at[pl.ds(row0, _C)], wsems[b]).start()

    def wait_wb(b):
        pltpu.make_async_copy(
            buf_v.at[b], out_hbm.at[pl.ds(0, _C)], wsems[b]).wait()

    def chunk(g, b, wait_writeback, prefetch):
        wait_fetch(b)
        start_wb(g, b)
        b2 = (b + 2) % _NBUF
        if wait_writeback:
            wait_wb(b2)
        if prefetch:
            start_fetch(g + 2, b2)

    start_fetch(0, 0)
    start_fetch(1, 1)
    chunk(0, 0, wait_writeback=False, prefetch=True)
    chunk(1, 1, wait_writeback=False, prefetch=True)
    chunk(2, 2, wait_writeback=True, prefetch=True)
    chunk(3, 3, wait_writeback=True, prefetch=True)

    def round_body(i, _):
        g0 = i * _NBUF
        chunk(g0, 0, wait_writeback=True, prefetch=True)
        chunk(g0 + 1, 1, wait_writeback=True, prefetch=True)
        chunk(g0 + 2, 2, wait_writeback=True, prefetch=True)
        chunk(g0 + 3, 3, wait_writeback=True, prefetch=True)
        return 0

    lax.fori_loop(1, _NCH // _NBUF - 1, round_body, 0)
    chunk(_NCH - 4, 0, wait_writeback=True, prefetch=True)
    chunk(_NCH - 3, 1, wait_writeback=True, prefetch=True)
    chunk(_NCH - 2, 2, wait_writeback=False, prefetch=False)
    chunk(_NCH - 1, 3, wait_writeback=False, prefetch=False)
    wait_wb(0)
    wait_wb(1)
    wait_wb(2)
    wait_wb(3)


_gather_kernel = functools.partial(
    pl.kernel,
    mesh=plsc.VectorSubcoreMesh(core_axis_name="c", subcore_axis_name="s"),
    out_type=jax.ShapeDtypeStruct((_R, _D), jnp.float32),
    scratch_types=[
        pltpu.VMEM((_RPW,), jnp.int32),
        pltpu.VMEM((_NBUF, _C, _D), jnp.float32),
        pltpu.SemaphoreType.DMA,
        pltpu.SemaphoreType.DMA,
        pltpu.SemaphoreType.DMA,
        pltpu.SemaphoreType.DMA,
        pltpu.SemaphoreType.DMA,
        pltpu.SemaphoreType.DMA,
        pltpu.SemaphoreType.DMA,
        pltpu.SemaphoreType.DMA,
    ],
)(_gather_body)


def _ln_body(g_ref, p_ref, gam_ref, bet_ref, o_ref):
    y = g_ref[...] * _SCALE + p_ref[...]
    mean = jnp.mean(y, axis=-1, keepdims=True)
    yc = y - mean
    var = jnp.mean(yc * yc, axis=-1, keepdims=True)
    o_ref[...] = yc * lax.rsqrt(var + _EPS) * gam_ref[0] + bet_ref[0]


_SBLK = _S // _RB

_ln_call = pl.pallas_call(
    _ln_body,
    grid=(_R // _RB,),
    in_specs=[
        pl.BlockSpec((_RB, _D), lambda i: ((i % _B) * _SBLK + i // _B, 0)),
        pl.BlockSpec((_RB, _D), lambda i: (i // _B, 0)),
        pl.BlockSpec((1, _D), lambda i: (0, 0)),
        pl.BlockSpec((1, _D), lambda i: (0, 0)),
    ],
    out_specs=pl.BlockSpec((_RB, _D), lambda i: (i // _B, i % _B)),
    out_shape=jax.ShapeDtypeStruct((_S, _B * _D), jnp.float32),
    compiler_params=pltpu.CompilerParams(
        dimension_semantics=("parallel",)),
)


def kernel(src_tokens, tok_table, pos_table, ln_gamma, ln_beta):
    idx = src_tokens.reshape(-1)
    gathered = _gather_kernel(idx, tok_table)
    out = _ln_call(gathered, pos_table,
                   ln_gamma.reshape(1, _D), ln_beta.reshape(1, _D))
    return out.reshape(_S, _B, _D)


# speedup vs baseline: 1.8503x; 1.0116x over previous
import functools

import jax
import jax.numpy as jnp
from jax import lax
from jax.experimental import pallas as pl
from jax.experimental.pallas import tpu as pltpu
from jax.experimental.pallas import tpu_sc as plsc

_VOCAB = 100000
_D = 1024
_B = 4
_S = 8192
_SCALE = 32.0
_EPS = 1e-5

_R = _S * _B
_NW = 32
_RPW = _R // _NW
_C = 16
_NCH = _RPW // _C
_NBUF = 4

_RB = 2048


def _gather_body(idx_hbm, tok_hbm, out_hbm,
                 idx_v, buf_v,
                 gsem0, gsem1, gsem2, gsem3, wsem0, wsem1, wsem2, wsem3):
    wid = lax.axis_index("s") * 2 + lax.axis_index("c")
    base = wid * _RPW
    gsems = (gsem0, gsem1, gsem2, gsem3)
    wsems = (wsem0, wsem1, wsem2, wsem3)

    pltpu.sync_copy(idx_hbm.at[pl.ds(pl.multiple_of(base, _RPW), _RPW)], idx_v)

    def start_fetch(g, b):
        off = pl.ds(pl.multiple_of(g * _C, _C), _C)
        pltpu.make_async_copy(
            tok_hbm.at[idx_v.at[off]], buf_v.at[b], gsems[b]).start()

    def wait_fetch(b):
        pltpu.make_async_copy(
            tok_hbm.at[idx_v.at[pl.ds(0, _C)]], buf_v.at[b], gsems[b]).wait()

    def start_wb(g, b):
        row0 = pl.multiple_of(base + g * _C, _C)
        pltpu.make_async_copy(
            buf_v.at[b], out_hbm.at[pl.ds(row0, _C)], wsems[b]).start()

    def wait_wb(b):
        pltpu.make_async_copy(
            buf_v.at[b], out_hbm.at[pl.ds(0, _C)], wsems[b]).wait()

    def chunk(g, b, wait_writeback, prefetch):
        wait_fetch(b)
        start_wb(g, b)
        b2 = (b + 2) % _NBUF
        if wait_writeback:
            wait_wb(b2)
        if prefetch:
            start_fetch(g + 2, b2)

    start_fetch(0, 0)
    start_fetch(1, 1)
    chunk(0, 0, wait_writeback=False, prefetch=True)
    chunk(1, 1, wait_writeback=False, prefetch=True)
    chunk(2, 2, wait_writeback=True, prefetch=True)
    chunk(3, 3, wait_writeback=True, prefetch=True)

    def round_body(i, _):
        g0 = i * _NBUF
        chunk(g0, 0, wait_writeback=True, prefetch=True)
        chunk(g0 + 1, 1, wait_writeback=True, prefetch=True)
        chunk(g0 + 2, 2, wait_writeback=True, prefetch=True)
        chunk(g0 + 3, 3, wait_writeback=True, prefetch=True)
        return 0

    lax.fori_loop(1, _NCH // _NBUF - 1, round_body, 0)
    chunk(_NCH - 4, 0, wait_writeback=True, prefetch=True)
    chunk(_NCH - 3, 1, wait_writeback=True, prefetch=True)
    chunk(_NCH - 2, 2, wait_writeback=False, prefetch=False)
    chunk(_NCH - 1, 3, wait_writeback=False, prefetch=False)
    wait_wb(0)
    wait_wb(1)
    wait_wb(2)
    wait_wb(3)


_gather_kernel = functools.partial(
    pl.kernel,
    mesh=plsc.VectorSubcoreMesh(core_axis_name="c", subcore_axis_name="s"),
    out_type=jax.ShapeDtypeStruct((_R, _D), jnp.float32),
    scratch_types=[
        pltpu.VMEM((_RPW,), jnp.int32),
        pltpu.VMEM((_NBUF, _C, _D), jnp.float32),
        pltpu.SemaphoreType.DMA,
        pltpu.SemaphoreType.DMA,
        pltpu.SemaphoreType.DMA,
        pltpu.SemaphoreType.DMA,
        pltpu.SemaphoreType.DMA,
        pltpu.SemaphoreType.DMA,
        pltpu.SemaphoreType.DMA,
        pltpu.SemaphoreType.DMA,
    ],
)(_gather_body)


def _ln_body(g_ref, p_ref, gam_ref, bet_ref, o_ref):
    y = g_ref[...] * _SCALE + p_ref[...]
    mean = jnp.mean(y, axis=-1, keepdims=True)
    yc = y - mean
    var = jnp.mean(yc * yc, axis=-1, keepdims=True)
    o_ref[...] = yc * lax.rsqrt(var + _EPS) * gam_ref[0] + bet_ref[0]


_SBLK = _S // _RB

_ln_call = pl.pallas_call(
    _ln_body,
    grid=(_R // _RB,),
    in_specs=[
        pl.BlockSpec((_RB, _D), lambda i: ((i % _B) * _SBLK + i // _B, 0)),
        pl.BlockSpec((_RB, _D), lambda i: (i // _B, 0)),
        pl.BlockSpec((1, _D), lambda i: (0, 0)),
        pl.BlockSpec((1, _D), lambda i: (0, 0)),
    ],
    out_specs=pl.BlockSpec((_RB, _D), lambda i: (i // _B, i % _B)),
    out_shape=jax.ShapeDtypeStruct((_S, _B * _D), jnp.float32),
    compiler_params=pltpu.CompilerParams(
        dimension_semantics=("parallel",)),
)


def kernel(src_tokens, tok_table, pos_table, ln_gamma, ln_beta):
    idx = src_tokens.reshape(-1)
    gathered = _gather_kernel(idx, tok_table)
    out = _ln_call(gathered, pos_table,
                   ln_gamma.reshape(1, _D), ln_beta.reshape(1, _D))
    return out.reshape(_S, _B, _D)
